# baseline (speedup 1.0000x reference)
import jax
import jax.numpy as jnp
from jax.experimental import pallas as pl
from jax.experimental.pallas import tpu as pltpu

_BI = 400


def _gcn_kernel(a0_ref, a1_ref, xfull_ref, w_ref, b_ref, out_ref, s_ref):
    i = pl.program_id(0)
    h = _BI // 2

    @pl.when(i == 0)
    def _():
        s_ref[...] = jnp.dot(
            xfull_ref[...].astype(jnp.bfloat16),
            w_ref[...].astype(jnp.bfloat16),
            preferred_element_type=jnp.float32,
        ).astype(jnp.bfloat16)

    acc0 = jnp.dot(
        a0_ref[...].astype(jnp.bfloat16),
        s_ref[...],
        preferred_element_type=jnp.float32,
    )
    acc1 = jnp.dot(
        a1_ref[...].astype(jnp.bfloat16),
        s_ref[...],
        preferred_element_type=jnp.float32,
    )
    y0 = acc0 + b_ref[...]
    y0 = jnp.where(y0 >= 0, y0, 0.01 * y0)
    out_ref[:h, :] = y0 + xfull_ref[pl.ds(i * _BI, h), :]
    y1 = acc1 + b_ref[...]
    y1 = jnp.where(y1 >= 0, y1, 0.01 * y1)
    out_ref[h:, :] = y1 + xfull_ref[pl.ds(i * _BI + h, h), :]


def kernel(x, adj, W, b):
    n, d = x.shape
    b2 = b.reshape(1, d).astype(jnp.float32)
    out = pl.pallas_call(
        _gcn_kernel,
        grid=(n // _BI,),
        in_specs=[
            pl.BlockSpec((_BI // 2, n), lambda i: (2 * i, 0)),
            pl.BlockSpec((_BI // 2, n), lambda i: (2 * i + 1, 0)),
            pl.BlockSpec((n, d), lambda i: (0, 0)),
            pl.BlockSpec((d, d), lambda i: (0, 0)),
            pl.BlockSpec((1, d), lambda i: (0, 0)),
        ],
        out_specs=pl.BlockSpec((_BI, d), lambda i: (i, 0)),
        out_shape=jax.ShapeDtypeStruct((n, d), jnp.float32),
        scratch_shapes=[pltpu.VMEM((n, d), jnp.bfloat16)],
    )(adj, adj, x, W, b2)
    return out
